# baseline (speedup 1.0000x reference)
import functools

import jax
import jax.numpy as jnp
from jax.experimental import pallas as pl
from jax.experimental.pallas import tpu as pltpu


def _fused_kernel(
    m_atoms_ref,
    m_bonds_ref,
    p_atoms_ref,
    p_edges_ref,
    m_edges_ref,
    W_m1a_ref, W_m1b_ref, b_m1_ref,
    W_m2a_ref, W_m2b_ref, W_m2c_ref, b_m2_ref,
    W_p1_ref, b_p1_ref,
    W_p2_ref, b_p2_ref,
    W_gopa_ref, W_gopb_ref, b_gop_ref,
    W_gopp_ref, b_gopp_ref,
    W_fc1a_ref, W_fc1b_ref, b_fc1_ref,
    W_fc2_ref, b_fc2_ref,
    out_ref,
):
    BB = m_atoms_ref.shape[0]
    Nm = m_atoms_ref.shape[1]
    D = m_edges_ref.shape[2]
    f32 = jnp.float32
    hi = jax.lax.Precision.HIGHEST

    for i in range(BB):
        pa = p_atoms_ref[i]
        pe = p_edges_ref[i]
        t1 = jnp.dot(pe, pa, preferred_element_type=f32)
        x1 = jax.nn.relu(jnp.dot(t1, W_p1_ref[:, :], preferred_element_type=f32)
                         + b_p1_ref[:, :])
        t2 = jnp.dot(pe, x1, preferred_element_type=f32)
        x2 = jax.nn.relu(jnp.dot(t2, W_p2_ref[:, :], preferred_element_type=f32)
                         + b_p2_ref[:, :])
        tp = jnp.tanh(jnp.dot(x2, W_gopp_ref[:, :], preferred_element_type=f32)
                      + b_gopp_ref[:, :])
        fp_p = jnp.sum(tp, axis=0, keepdims=True)

        bsum = m_bonds_ref[i, 0]
        for d in range(1, m_bonds_ref.shape[1]):
            bsum = bsum + m_bonds_ref[i, d]

        e = m_edges_ref[i]
        iota = jax.lax.broadcasted_iota(jnp.int32, (Nm, Nm), 1)
        A = jnp.zeros((Nm, Nm), dtype=f32)
        for d in range(D):
            col = jax.lax.slice(e, (0, d), (Nm, d + 1))
            A = A + (col == iota).astype(f32)

        atoms = m_atoms_ref[i]
        s1 = jnp.dot(A, atoms, preferred_element_type=f32, precision=hi) + atoms
        hm1 = jax.nn.relu(
            jnp.dot(s1, W_m1a_ref[:, :], preferred_element_type=f32)
            + jnp.dot(bsum, W_m1b_ref[:, :], preferred_element_type=f32)
            + b_m1_ref[:, :])

        s2h = jnp.dot(A, hm1, preferred_element_type=f32, precision=hi) + hm1
        s2b = jnp.dot(A, bsum, preferred_element_type=f32, precision=hi) + bsum
        hm2 = jax.nn.relu(
            jnp.dot(s2h, W_m2a_ref[:, :], preferred_element_type=f32)
            + jnp.dot(s2b, W_m2b_ref[:, :], preferred_element_type=f32)
            + jnp.dot(bsum, W_m2c_ref[:, :], preferred_element_type=f32)
            + b_m2_ref[:, :])

        tm = jnp.tanh(
            jnp.dot(hm2, W_gopa_ref[:, :], preferred_element_type=f32)
            + jnp.dot(bsum, W_gopb_ref[:, :], preferred_element_type=f32)
            + b_gop_ref[:, :])
        fp_m = jnp.sum(tm, axis=0, keepdims=True)

        inter = jax.nn.sigmoid(
            jnp.dot(fp_m, W_fc1a_ref[:, :], preferred_element_type=f32)
            + jnp.dot(fp_p, W_fc1b_ref[:, :], preferred_element_type=f32)
            + b_fc1_ref[:, :])
        logits = jnp.dot(inter, W_fc2_ref[:, :], preferred_element_type=f32) \
            + b_fc2_ref[:, :]
        m = jnp.max(logits, axis=1, keepdims=True)
        ex = jnp.exp(logits - m)
        out_ref[i] = ex / jnp.sum(ex, axis=1, keepdims=True)


@jax.jit
def kernel(m_atoms, m_bonds, p_atoms, p_edges,
           W_m1, b_m1, W_m2, b_m2, W_p1, b_p1, W_p2, b_p2,
           W_gop, b_gop, W_gopp, b_gopp, W_fc1, b_fc1, W_fc2, b_fc2,
           m_edges):
    B, Nm, Fa = m_atoms.shape
    D = m_edges.shape[2]
    Np = p_atoms.shape[1]
    H = W_m1.shape[1]

    m_bonds_t = jnp.transpose(m_bonds, (0, 2, 1, 3))
    m_edges32 = m_edges.astype(jnp.int32)
    row = lambda v: v.reshape(1, -1)

    W_m1a, W_m1b = W_m1[:Fa], W_m1[Fa:]
    W_m2a, W_m2b, W_m2c = W_m2[:H], W_m2[H:H + 6], W_m2[H + 6:]
    W_gopa, W_gopb = W_gop[:H], W_gop[H:]
    W_fc1a, W_fc1b = W_fc1[:H], W_fc1[H:]

    BB = 2

    def whole(x):
        return pl.BlockSpec(x.shape, lambda b: (0,) * x.ndim)

    batch3 = lambda x: pl.BlockSpec((BB,) + x.shape[1:],
                                    lambda b: (b,) + (0,) * (x.ndim - 1))

    operands = [
        m_atoms, m_bonds_t, p_atoms, p_edges, m_edges32,
        W_m1a, W_m1b, row(b_m1),
        W_m2a, W_m2b, W_m2c, row(b_m2),
        W_p1, row(b_p1), W_p2, row(b_p2),
        W_gopa, W_gopb, row(b_gop),
        W_gopp, row(b_gopp),
        W_fc1a, W_fc1b, row(b_fc1),
        W_fc2, row(b_fc2),
    ]
    in_specs = [batch3(m_atoms), batch3(m_bonds_t), batch3(p_atoms),
                batch3(p_edges), batch3(m_edges32)] + \
               [whole(x) for x in operands[5:]]

    out = pl.pallas_call(
        _fused_kernel,
        grid=(B // BB,),
        in_specs=in_specs,
        out_specs=pl.BlockSpec((BB, 1, 2), lambda b: (b, 0, 0)),
        out_shape=jax.ShapeDtypeStruct((B, 1, 2), jnp.float32),
        compiler_params=pltpu.CompilerParams(
            dimension_semantics=("parallel",)),
    )(*operands)
    return out.reshape(B, 2)


# speedup vs baseline: 1.0638x; 1.0638x over previous
import jax
import jax.numpy as jnp
from jax.experimental import pallas as pl
from jax.experimental.pallas import tpu as pltpu


def _fused_kernel(
    m_atoms_ref,
    m_bonds_ref,
    p_atoms_ref,
    p_edges_ref,
    m_edges_ref,
    W_m1_ref, b_m1_ref,
    W_m2_ref, b_m2_ref,
    W_p1_ref, b_p1_ref,
    W_p2_ref, b_p2_ref,
    W_gop_ref, b_gop_ref,
    W_gopp_ref, b_gopp_ref,
    W_fc1_ref, b_fc1_ref,
    W_fc2_ref, b_fc2_ref,
    out_ref,
):
    BB, Nm, Fa = m_atoms_ref.shape
    D = m_edges_ref.shape[2]
    H = W_m1_ref.shape[1]
    f32 = jnp.float32
    hi = jax.lax.Precision.HIGHEST

    W_m1a = W_m1_ref[0:Fa, :]
    W_m1b = W_m1_ref[Fa:Fa + 6, :]
    W_m2a = W_m2_ref[0:H, :]
    W_m2b = W_m2_ref[H:H + 6, :]
    W_m2c = W_m2_ref[H + 6:H + 12, :]
    W_gopa = W_gop_ref[0:H, :]
    W_gopb = W_gop_ref[H:H + 6, :]
    W_fc1a = W_fc1_ref[0:H, :]
    W_fc1b = W_fc1_ref[H:2 * H, :]
    b_m1 = b_m1_ref[:].reshape(1, -1)
    b_m2 = b_m2_ref[:].reshape(1, -1)
    b_p1 = b_p1_ref[:].reshape(1, -1)
    b_p2 = b_p2_ref[:].reshape(1, -1)
    b_gop = b_gop_ref[:].reshape(1, -1)
    b_gopp = b_gopp_ref[:].reshape(1, -1)
    b_fc1 = b_fc1_ref[:].reshape(1, -1)
    b_fc2 = b_fc2_ref[:].reshape(1, -1)

    for i in range(BB):
        pa = p_atoms_ref[i]
        pe = p_edges_ref[i]
        t1 = jnp.dot(pe, pa, preferred_element_type=f32)
        x1 = jax.nn.relu(jnp.dot(t1, W_p1_ref[:, :], preferred_element_type=f32)
                         + b_p1)
        t2 = jnp.dot(pe, x1, preferred_element_type=f32)
        x2 = jax.nn.relu(jnp.dot(t2, W_p2_ref[:, :], preferred_element_type=f32)
                         + b_p2)
        tp = jnp.tanh(jnp.dot(x2, W_gopp_ref[:, :], preferred_element_type=f32)
                      + b_gopp)
        fp_p = jnp.sum(tp, axis=0, keepdims=True)

        bsum = jnp.sum(m_bonds_ref[i], axis=1)

        e = m_edges_ref[i]
        iota = jax.lax.broadcasted_iota(jnp.int32, (Nm, Nm), 1)
        A = jnp.zeros((Nm, Nm), dtype=f32)
        for d in range(D):
            col = jax.lax.slice(e, (0, d), (Nm, d + 1))
            A = A + (col == iota).astype(f32)

        atoms = m_atoms_ref[i]
        s1 = jnp.dot(A, atoms, preferred_element_type=f32, precision=hi) + atoms
        hm1 = jax.nn.relu(
            jnp.dot(s1, W_m1a, preferred_element_type=f32)
            + jnp.dot(bsum, W_m1b, preferred_element_type=f32)
            + b_m1)

        s2h = jnp.dot(A, hm1, preferred_element_type=f32, precision=hi) + hm1
        s2b = jnp.dot(A, bsum, preferred_element_type=f32, precision=hi) + bsum
        hm2 = jax.nn.relu(
            jnp.dot(s2h, W_m2a, preferred_element_type=f32)
            + jnp.dot(s2b, W_m2b, preferred_element_type=f32)
            + jnp.dot(bsum, W_m2c, preferred_element_type=f32)
            + b_m2)

        tm = jnp.tanh(
            jnp.dot(hm2, W_gopa, preferred_element_type=f32)
            + jnp.dot(bsum, W_gopb, preferred_element_type=f32)
            + b_gop)
        fp_m = jnp.sum(tm, axis=0, keepdims=True)

        inter = jax.nn.sigmoid(
            jnp.dot(fp_m, W_fc1a, preferred_element_type=f32)
            + jnp.dot(fp_p, W_fc1b, preferred_element_type=f32)
            + b_fc1)
        logits = jnp.dot(inter, W_fc2_ref[:, :], preferred_element_type=f32) \
            + b_fc2
        m = jnp.max(logits, axis=1, keepdims=True)
        ex = jnp.exp(logits - m)
        out_ref[i] = ex / jnp.sum(ex, axis=1, keepdims=True)


@jax.jit
def kernel(m_atoms, m_bonds, p_atoms, p_edges,
           W_m1, b_m1, W_m2, b_m2, W_p1, b_p1, W_p2, b_p2,
           W_gop, b_gop, W_gopp, b_gopp, W_fc1, b_fc1, W_fc2, b_fc2,
           m_edges):
    B = m_atoms.shape[0]
    m_edges32 = m_edges.astype(jnp.int32)

    BB = 2

    def whole(x):
        return pl.BlockSpec(x.shape, lambda b: (0,) * x.ndim)

    batch3 = lambda x: pl.BlockSpec((BB,) + x.shape[1:],
                                    lambda b: (b,) + (0,) * (x.ndim - 1))

    operands = [
        m_atoms, m_bonds, p_atoms, p_edges, m_edges32,
        W_m1, b_m1, W_m2, b_m2,
        W_p1, b_p1, W_p2, b_p2,
        W_gop, b_gop, W_gopp, b_gopp,
        W_fc1, b_fc1, W_fc2, b_fc2,
    ]
    in_specs = [batch3(m_atoms), batch3(m_bonds), batch3(p_atoms),
                batch3(p_edges), batch3(m_edges32)] + \
               [whole(x) for x in operands[5:]]

    out = pl.pallas_call(
        _fused_kernel,
        grid=(B // BB,),
        in_specs=in_specs,
        out_specs=pl.BlockSpec((BB, 1, 2), lambda b: (b, 0, 0)),
        out_shape=jax.ShapeDtypeStruct((B, 1, 2), jnp.float32),
        compiler_params=pltpu.CompilerParams(
            dimension_semantics=("parallel",)),
    )(*operands)
    return out.reshape(B, 2)


# speedup vs baseline: 1.1850x; 1.1140x over previous
import jax
import jax.numpy as jnp
from jax.experimental import pallas as pl
from jax.experimental.pallas import tpu as pltpu


def _fused_kernel(
    m_atoms_ref,
    m_bonds_ref,
    p_atoms_ref,
    p_edges_ref,
    m_edges_ref,
    W_m1_ref, b_m1_ref,
    W_m2_ref, b_m2_ref,
    W_p1_ref, b_p1_ref,
    W_p2_ref, b_p2_ref,
    W_gop_ref, b_gop_ref,
    W_gopp_ref, b_gopp_ref,
    W_fc1_ref, b_fc1_ref,
    W_fc2_ref, b_fc2_ref,
    out_ref,
    fpp_ref,
):
    B, Nm, Fa = m_atoms_ref.shape
    D = m_edges_ref.shape[2]
    BB = p_atoms_ref.shape[0]
    H = W_m1_ref.shape[1]
    f32 = jnp.float32
    hi = jax.lax.Precision.HIGHEST
    pid = pl.program_id(0)
    nsteps = pl.num_programs(0)

    b_p1 = b_p1_ref[:].reshape(1, -1)
    b_p2 = b_p2_ref[:].reshape(1, -1)
    b_gopp = b_gopp_ref[:].reshape(1, -1)

    for i in range(BB):
        pa = p_atoms_ref[i]
        pe = p_edges_ref[i]
        t1 = jnp.dot(pe, pa, preferred_element_type=f32)
        x1 = jax.nn.relu(jnp.dot(t1, W_p1_ref[:, :], preferred_element_type=f32)
                         + b_p1)
        t2 = jnp.dot(pe, x1, preferred_element_type=f32)
        x2 = jax.nn.relu(jnp.dot(t2, W_p2_ref[:, :], preferred_element_type=f32)
                         + b_p2)
        tp = jnp.tanh(jnp.dot(x2, W_gopp_ref[:, :], preferred_element_type=f32)
                      + b_gopp)
        fp_p = jnp.sum(tp, axis=0, keepdims=True)
        fpp_ref[pl.ds(pid * BB + i, 1), :] = fp_p

    @pl.when(pid == nsteps - 1)
    def _tail():
        W_m1a = W_m1_ref[0:Fa, :]
        W_m1b = W_m1_ref[Fa:Fa + 6, :]
        W_m2a = W_m2_ref[0:H, :]
        W_m2b = W_m2_ref[H:H + 6, :]
        W_m2c = W_m2_ref[H + 6:H + 12, :]
        W_gopa = W_gop_ref[0:H, :]
        W_gopb = W_gop_ref[H:H + 6, :]
        W_fc1a = W_fc1_ref[0:H, :]
        W_fc1b = W_fc1_ref[H:2 * H, :]
        b_m1 = b_m1_ref[:].reshape(1, -1)
        b_m2 = b_m2_ref[:].reshape(1, -1)
        b_gop = b_gop_ref[:].reshape(1, -1)
        b_fc1 = b_fc1_ref[:].reshape(1, -1)
        b_fc2 = b_fc2_ref[:].reshape(1, -1)

        iota = jax.lax.broadcasted_iota(jnp.int32, (Nm, Nm), 1)

        def adj(b):
            e = m_edges_ref[b]
            A = jnp.zeros((Nm, Nm), dtype=f32)
            for d in range(D):
                col = jax.lax.slice(e, (0, d), (Nm, d + 1))
                A = A + (col == iota).astype(f32)
            return A

        bsum_l, s1_l = [], []
        for b in range(B):
            atoms = m_atoms_ref[b]
            bsum_l.append(jnp.sum(m_bonds_ref[b], axis=1))
            s1_l.append(jnp.dot(adj(b), atoms, preferred_element_type=f32,
                                precision=hi) + atoms)
        bsum = jnp.concatenate(bsum_l, axis=0)
        s1 = jnp.concatenate(s1_l, axis=0)
        hm1 = jax.nn.relu(
            jnp.dot(s1, W_m1a, preferred_element_type=f32)
            + jnp.dot(bsum, W_m1b, preferred_element_type=f32)
            + b_m1)

        s2h_l, s2b_l = [], []
        for b in range(B):
            A = adj(b)
            hm1_b = jax.lax.slice(hm1, (b * Nm, 0), ((b + 1) * Nm, H))
            bsum_b = bsum_l[b]
            s2h_l.append(jnp.dot(A, hm1_b, preferred_element_type=f32,
                                 precision=hi) + hm1_b)
            s2b_l.append(jnp.dot(A, bsum_b, preferred_element_type=f32,
                                 precision=hi) + bsum_b)
        s2h = jnp.concatenate(s2h_l, axis=0)
        s2b = jnp.concatenate(s2b_l, axis=0)
        hm2 = jax.nn.relu(
            jnp.dot(s2h, W_m2a, preferred_element_type=f32)
            + jnp.dot(s2b, W_m2b, preferred_element_type=f32)
            + jnp.dot(bsum, W_m2c, preferred_element_type=f32)
            + b_m2)

        tm = jnp.tanh(
            jnp.dot(hm2, W_gopa, preferred_element_type=f32)
            + jnp.dot(bsum, W_gopb, preferred_element_type=f32)
            + b_gop)

        rows = jax.lax.broadcasted_iota(jnp.int32, (B, B * Nm), 0)
        cols = jax.lax.broadcasted_iota(jnp.int32, (B, B * Nm), 1)
        S = (cols // Nm == rows).astype(f32)
        fp_m = jnp.dot(S, tm, preferred_element_type=f32, precision=hi)

        fp_p = fpp_ref[:, :]
        inter = jax.nn.sigmoid(
            jnp.dot(fp_m, W_fc1a, preferred_element_type=f32)
            + jnp.dot(fp_p, W_fc1b, preferred_element_type=f32)
            + b_fc1)
        logits = jnp.dot(inter, W_fc2_ref[:, :], preferred_element_type=f32) \
            + b_fc2
        mx = jnp.max(logits, axis=1, keepdims=True)
        ex = jnp.exp(logits - mx)
        probs = ex / jnp.sum(ex, axis=1, keepdims=True)
        out_ref[:, :, :] = probs.reshape(B, 1, 2)


@jax.jit
def kernel(m_atoms, m_bonds, p_atoms, p_edges,
           W_m1, b_m1, W_m2, b_m2, W_p1, b_p1, W_p2, b_p2,
           W_gop, b_gop, W_gopp, b_gopp, W_fc1, b_fc1, W_fc2, b_fc2,
           m_edges):
    B = m_atoms.shape[0]
    m_edges32 = m_edges.astype(jnp.int32)

    BB = 2

    def whole(x):
        return pl.BlockSpec(x.shape, lambda b: (0,) * x.ndim)

    stream = lambda x: pl.BlockSpec((BB,) + x.shape[1:],
                                    lambda b: (b,) + (0,) * (x.ndim - 1))

    operands = [
        m_atoms, m_bonds, p_atoms, p_edges, m_edges32,
        W_m1, b_m1, W_m2, b_m2,
        W_p1, b_p1, W_p2, b_p2,
        W_gop, b_gop, W_gopp, b_gopp,
        W_fc1, b_fc1, W_fc2, b_fc2,
    ]
    in_specs = [whole(m_atoms), whole(m_bonds), stream(p_atoms),
                stream(p_edges), whole(m_edges32)] + \
               [whole(x) for x in operands[5:]]

    out = pl.pallas_call(
        _fused_kernel,
        grid=(B // BB,),
        in_specs=in_specs,
        out_specs=pl.BlockSpec((B, 1, 2), lambda b: (0, 0, 0)),
        out_shape=jax.ShapeDtypeStruct((B, 1, 2), jnp.float32),
        scratch_shapes=[pltpu.VMEM((B, 128), jnp.float32)],
        compiler_params=pltpu.CompilerParams(
            dimension_semantics=("arbitrary",)),
    )(*operands)
    return out.reshape(B, 2)
